# baseline (speedup 1.0000x reference)
import functools

import jax
import jax.numpy as jnp
from jax import lax
from jax.experimental import pallas as pl
from jax.experimental.pallas import tpu as pltpu
from jax.experimental.pallas import tpu_sc as plsc

B = 16384
L = 200
EMB = 32
NCLS = 16

NC = 2
NS = 16
NW = NC * NS
D_TILE = B // NW
HALF = L // 2
NCHUNK = 2
DC = D_TILE // NCHUNK
NBUF = 8

PROJ_RB = 8192


def _tc_proj(embed, W):
    V2 = embed.shape[0]

    def body(e_ref, w_ref, o_ref):
        o_ref[...] = lax.dot_general(
            e_ref[...],
            w_ref[...],
            dimension_numbers=(((1,), (1,)), ((), ())),
            preferred_element_type=jnp.float32,
        )

    return pl.pallas_call(
        body,
        grid=(pl.cdiv(V2, PROJ_RB),),
        in_specs=[
            pl.BlockSpec((PROJ_RB, EMB), lambda i: (i, 0)),
            pl.BlockSpec((NCLS, EMB), lambda i: (0, 0)),
        ],
        out_specs=pl.BlockSpec((PROJ_RB, NCLS), lambda i: (i, 0)),
        out_shape=jax.ShapeDtypeStruct((V2, NCLS), jnp.float32),
    )(embed, W)


def _sc_pool(idx_hr, proj):
    mesh = plsc.VectorSubcoreMesh(core_axis_name="c", subcore_axis_name="s")

    @functools.partial(
        pl.kernel,
        mesh=mesh,
        out_type=jax.ShapeDtypeStruct((B, NCLS), jnp.float32),
        compiler_params=pltpu.CompilerParams(use_tc_tiling_on_sc=False),
        scratch_types=[
            pltpu.VMEM((DC * 2, HALF), jnp.int32),
            pltpu.VMEM((NBUF, L, NCLS), jnp.float32),
            pltpu.VMEM((D_TILE, NCLS), jnp.float32),
            pltpu.SemaphoreType.DMA((NBUF,)),
        ],
    )
    def k(idx_hbm, proj_hbm, out_hbm, idx_v, bufs, out_v, gsem):
        wid = lax.axis_index("s") * NC + lax.axis_index("c")
        hr_base = wid * (D_TILE * 2)

        def fire(d, par):
            pltpu.async_copy(
                proj_hbm.at[idx_v.at[2 * d]],
                bufs.at[par].at[pl.ds(0, HALF)],
                gsem.at[par],
            )
            pltpu.async_copy(
                proj_hbm.at[idx_v.at[2 * d + 1]],
                bufs.at[par].at[pl.ds(HALF, HALF)],
                gsem.at[par],
            )

        def wait_par(par):
            pltpu.make_async_copy(
                proj_hbm.at[idx_v.at[0]],
                bufs.at[par].at[pl.ds(0, HALF)],
                gsem.at[par],
            ).wait()
            pltpu.make_async_copy(
                proj_hbm.at[idx_v.at[0]],
                bufs.at[par].at[pl.ds(HALF, HALF)],
                gsem.at[par],
            ).wait()

        def sum_rows(buf):
            def body(i, a):
                for u in range(8):
                    r = i * 8 + u
                    a = a + buf[r, pl.ds(0, 16)]
                return a

            z = jnp.zeros((16,), jnp.float32)
            return lax.fori_loop(0, L // 8, body, z, unroll=False)

        def chunk_body(c, _):
            hr0 = hr_base + c * (DC * 2)
            pltpu.sync_copy(idx_hbm.at[pl.ds(hr0, DC * 2)], idx_v)
            for par in range(NBUF):
                fire(par, par)

            def group_body(p, _):
                for par in range(NBUF):
                    d = NBUF * p + par
                    wait_par(par)
                    a = sum_rows(bufs.at[par])
                    nd = d + NBUF

                    @pl.when(nd < DC)
                    def _():
                        fire(nd, par)

                    out_v[c * DC + d, pl.ds(0, 16)] = a
                return 0

            lax.fori_loop(0, DC // NBUF, group_body, 0, unroll=False)
            return 0

        lax.fori_loop(0, NCHUNK, chunk_body, 0, unroll=False)
        pltpu.sync_copy(out_v, out_hbm.at[pl.ds(wid * D_TILE, D_TILE)])

    return k(idx_hr, proj)


def _tc_tail(pooled, len_f, b2):
    def body(p_ref, l_ref, b_ref, o_ref):
        o_ref[...] = p_ref[...] / l_ref[...] + b_ref[...]

    return pl.pallas_call(
        body,
        out_shape=jax.ShapeDtypeStruct((B, NCLS), jnp.float32),
    )(pooled, len_f, b2)


def kernel(data, length, embed, W, b):
    idx_hr = data.astype(jnp.int32).reshape(B * 2, HALF)
    len_f = length.astype(jnp.float32).reshape(B, 1)
    proj = _tc_proj(embed, W)
    pooled = _sc_pool(idx_hr, proj)
    return _tc_tail(pooled, len_f, b.reshape(1, NCLS))


# speedup vs baseline: 1.6606x; 1.6606x over previous
import functools

import jax
import jax.numpy as jnp
from jax import lax
from jax.experimental import pallas as pl
from jax.experimental.pallas import tpu as pltpu
from jax.experimental.pallas import tpu_sc as plsc

B = 16384
L = 200
EMB = 32
NCLS = 16

NC = 2
NS = 16
NW = NC * NS
D_TILE = B // NW
H0 = 104
H1 = L - H0
NCHUNK = 2
DC = D_TILE // NCHUNK
NBUF = 8


def _sc_pool(data, embed):
    mesh = plsc.VectorSubcoreMesh(core_axis_name="c", subcore_axis_name="s")

    @functools.partial(
        pl.kernel,
        mesh=mesh,
        out_type=jax.ShapeDtypeStruct((B, EMB), jnp.float32),
        compiler_params=pltpu.CompilerParams(use_tc_tiling_on_sc=False),
        scratch_types=[
            pltpu.VMEM((DC, L), jnp.int32),
            pltpu.VMEM((NBUF, L, EMB), jnp.float32),
            pltpu.VMEM((D_TILE, EMB), jnp.float32),
            pltpu.SemaphoreType.DMA((NBUF,)),
        ],
    )
    def k(data_hbm, embed_hbm, out_hbm, idx_v, bufs, out_v, gsem):
        wid = lax.axis_index("s") * NC + lax.axis_index("c")
        doc_base = wid * D_TILE

        def fire(d, par):
            pltpu.async_copy(
                embed_hbm.at[idx_v.at[d, pl.ds(0, H0)]],
                bufs.at[par].at[pl.ds(0, H0)],
                gsem.at[par],
            )
            pltpu.async_copy(
                embed_hbm.at[idx_v.at[d, pl.ds(H0, H1)]],
                bufs.at[par].at[pl.ds(H0, H1)],
                gsem.at[par],
            )

        def wait_par(par):
            pltpu.make_async_copy(
                embed_hbm.at[idx_v.at[0, pl.ds(0, H0)]],
                bufs.at[par].at[pl.ds(0, H0)],
                gsem.at[par],
            ).wait()
            pltpu.make_async_copy(
                embed_hbm.at[idx_v.at[0, pl.ds(H0, H1)]],
                bufs.at[par].at[pl.ds(H0, H1)],
                gsem.at[par],
            ).wait()

        def sum_rows(buf):
            def body(i, accs):
                a0, a1 = accs
                for u in range(8):
                    r = i * 8 + u
                    a0 = a0 + buf[r, pl.ds(0, 16)]
                    a1 = a1 + buf[r, pl.ds(16, 16)]
                return a0, a1

            z = jnp.zeros((16,), jnp.float32)
            return lax.fori_loop(0, L // 8, body, (z, z), unroll=False)

        def chunk_body(c, _):
            pltpu.sync_copy(data_hbm.at[pl.ds(doc_base + c * DC, DC)], idx_v)
            for par in range(NBUF):
                fire(par, par)

            def group_body(p, _):
                for par in range(NBUF):
                    d = NBUF * p + par
                    wait_par(par)
                    a0, a1 = sum_rows(bufs.at[par])
                    nd = d + NBUF

                    @pl.when(nd < DC)
                    def _():
                        fire(nd, par)

                    row = c * DC + d
                    out_v[row, pl.ds(0, 16)] = a0
                    out_v[row, pl.ds(16, 16)] = a1
                return 0

            lax.fori_loop(0, DC // NBUF, group_body, 0, unroll=False)
            return 0

        lax.fori_loop(0, NCHUNK, chunk_body, 0, unroll=False)
        pltpu.sync_copy(out_v, out_hbm.at[pl.ds(wid * D_TILE, D_TILE)])

    return k(data, embed)


def _tc_head(pooled, len_f, W, b2):
    def body(p_ref, l_ref, w_ref, b_ref, o_ref):
        x = p_ref[...] / l_ref[...]
        o_ref[...] = (
            lax.dot_general(
                x,
                w_ref[...],
                dimension_numbers=(((1,), (1,)), ((), ())),
                preferred_element_type=jnp.float32,
            )
            + b_ref[...]
        )

    return pl.pallas_call(
        body,
        out_shape=jax.ShapeDtypeStruct((B, NCLS), jnp.float32),
    )(pooled, len_f, W, b2)


def kernel(data, length, embed, W, b):
    len_f = length.astype(jnp.float32).reshape(B, 1)
    pooled = _sc_pool(data.astype(jnp.int32), embed)
    return _tc_head(pooled, len_f, W, b.reshape(1, NCLS))


# speedup vs baseline: 2.2374x; 1.3474x over previous
import functools

import jax
import jax.numpy as jnp
from jax import lax
from jax.experimental import pallas as pl
from jax.experimental.pallas import tpu as pltpu
from jax.experimental.pallas import tpu_sc as plsc

B = 16384
L = 200
EMB = 32
NCLS = 16
V = 1000002

NC = 2
NS = 16
NW = NC * NS
D_TILE = B // NW
H0 = 104
H1 = L - H0
NCHUNK = 2
DC = D_TILE // NCHUNK
NBUF = 8

N = 131072
SH = 17
RB = 2048
NBI = N // RB
CMAX = (V - 1) // RB


def _tc_proj_pack(embed_t, W):
    def body(*refs):
        x_refs, w_ref, o_ref = refs[:8], refs[8], refs[9]
        for j in range(8):
            o_ref[:, 16 * j : 16 * (j + 1)] = lax.dot_general(
                x_refs[j][...],
                w_ref[...],
                dimension_numbers=(((0,), (1,)), ((), ())),
                preferred_element_type=jnp.float32,
            )

    def _xspec(j):
        return pl.BlockSpec(
            (EMB, RB), lambda i, j=j: (0, jnp.minimum(j * NBI + i, CMAX))
        )

    return pl.pallas_call(
        body,
        grid=(NBI,),
        in_specs=[_xspec(j) for j in range(8)]
        + [pl.BlockSpec((NCLS, EMB), lambda i: (0, 0))],
        out_specs=pl.BlockSpec((RB, 128), lambda i: (i, 0)),
        out_shape=jax.ShapeDtypeStruct((N, 128), jnp.float32),
    )(*([embed_t] * 8), W)


def _sc_pool(data, table):
    mesh = plsc.VectorSubcoreMesh(core_axis_name="c", subcore_axis_name="s")

    @functools.partial(
        pl.kernel,
        mesh=mesh,
        out_type=jax.ShapeDtypeStruct((B, NCLS), jnp.float32),
        compiler_params=pltpu.CompilerParams(use_tc_tiling_on_sc=False),
        scratch_types=[
            pltpu.VMEM((DC, L), jnp.int32),
            pltpu.VMEM((NBUF, L), jnp.int32),
            pltpu.VMEM((NBUF, L, NCLS), jnp.float32),
            pltpu.VMEM((D_TILE, NCLS), jnp.float32),
            pltpu.SemaphoreType.DMA((NBUF,)),
        ],
    )
    def k(data_hbm, tab_hbm, out_hbm, idx_v, idxq, bufs, out_v, gsem):
        wid = lax.axis_index("s") * NC + lax.axis_index("c")
        doc_base = wid * D_TILE

        offs = tuple(kk * 16 for kk in range(12)) + (L - 16,)

        def xform(d, par):
            for o in offs:
                v = idx_v[d, pl.ds(o, 16)]
                q = lax.shift_left(jnp.bitwise_and(v, N - 1), 3)
                q = jnp.bitwise_or(q, lax.shift_right_logical(v, SH))
                idxq[par, pl.ds(o, 16)] = q

        def fire(d, par):
            xform(d, par)
            pltpu.async_copy(
                tab_hbm.at[idxq.at[par, pl.ds(0, H0)]],
                bufs.at[par].at[pl.ds(0, H0)],
                gsem.at[par],
            )
            pltpu.async_copy(
                tab_hbm.at[idxq.at[par, pl.ds(H0, H1)]],
                bufs.at[par].at[pl.ds(H0, H1)],
                gsem.at[par],
            )

        def wait_par(par):
            pltpu.make_async_copy(
                tab_hbm.at[idxq.at[par, pl.ds(0, H0)]],
                bufs.at[par].at[pl.ds(0, H0)],
                gsem.at[par],
            ).wait()
            pltpu.make_async_copy(
                tab_hbm.at[idxq.at[par, pl.ds(0, H1)]],
                bufs.at[par].at[pl.ds(H0, H1)],
                gsem.at[par],
            ).wait()

        def sum_rows(buf):
            def body(i, a):
                for u in range(8):
                    a = a + buf[i * 8 + u, pl.ds(0, 16)]
                return a

            z = jnp.zeros((16,), jnp.float32)
            return lax.fori_loop(0, L // 8, body, z, unroll=False)

        def chunk_body(c, _):
            pltpu.sync_copy(data_hbm.at[pl.ds(doc_base + c * DC, DC)], idx_v)
            for par in range(NBUF):
                fire(par, par)

            def group_body(p, _):
                for par in range(NBUF):
                    d = NBUF * p + par
                    wait_par(par)
                    a = sum_rows(bufs.at[par])
                    nd = d + NBUF

                    @pl.when(nd < DC)
                    def _():
                        fire(nd, par)

                    out_v[c * DC + d, pl.ds(0, 16)] = a
                return 0

            lax.fori_loop(0, DC // NBUF, group_body, 0, unroll=False)
            return 0

        lax.fori_loop(0, NCHUNK, chunk_body, 0, unroll=False)
        pltpu.sync_copy(out_v, out_hbm.at[pl.ds(wid * D_TILE, D_TILE)])

    return k(data, table)


def _tc_tail(pooled, len_f, b2):
    def body(p_ref, l_ref, b_ref, o_ref):
        o_ref[...] = p_ref[...] / l_ref[...] + b_ref[...]

    return pl.pallas_call(
        body,
        out_shape=jax.ShapeDtypeStruct((B, NCLS), jnp.float32),
    )(pooled, len_f, b2)


def kernel(data, length, embed, W, b):
    len_f = length.astype(jnp.float32).reshape(B, 1)
    packed = _tc_proj_pack(embed.T, W)
    table = packed.reshape(8 * N, NCLS)
    pooled = _sc_pool(data.astype(jnp.int32), table)
    return _tc_tail(pooled, len_f, b.reshape(1, NCLS))


# speedup vs baseline: 3.4608x; 1.5468x over previous
import functools

import jax
import jax.numpy as jnp
from jax import lax
from jax.experimental import pallas as pl
from jax.experimental.pallas import tpu as pltpu
from jax.experimental.pallas import tpu_sc as plsc

B = 16384
L = 200
EMB = 32
NCLS = 16
V = 1000002

NC = 2
NS = 16
NW = NC * NS
D_TILE = B // NW
H0 = 104
H1 = L - H0
NCHUNK = 2
DC = D_TILE // NCHUNK
NBUF = 8

N = 131072
SH = 17
RB = 2048
NBI = N // RB
CMAX = (V - 1) // RB


def _tc_proj_pack(embed_t, W):
    def body(*refs):
        x_refs, w_ref, o_ref = refs[:8], refs[8], refs[9]
        xcat = jnp.concatenate([x_refs[j][...] for j in range(8)], axis=0)
        o_ref[...] = lax.dot_general(
            xcat,
            w_ref[...],
            dimension_numbers=(((0,), (0,)), ((), ())),
            preferred_element_type=jnp.float32,
        )

    def _xspec(j):
        return pl.BlockSpec(
            (EMB, RB), lambda i, j=j: (0, jnp.minimum(j * NBI + i, CMAX))
        )

    wbig = jnp.kron(jnp.eye(8, dtype=jnp.float32), W.T)
    return pl.pallas_call(
        body,
        grid=(NBI,),
        in_specs=[_xspec(j) for j in range(8)]
        + [pl.BlockSpec((8 * EMB, 128), lambda i: (0, 0))],
        out_specs=pl.BlockSpec((RB, 128), lambda i: (i, 0)),
        out_shape=jax.ShapeDtypeStruct((N, 128), jnp.float32),
    )(*([embed_t] * 8), wbig)


def _sc_pool(data, table):
    mesh = plsc.VectorSubcoreMesh(core_axis_name="c", subcore_axis_name="s")

    @functools.partial(
        pl.kernel,
        mesh=mesh,
        out_type=jax.ShapeDtypeStruct((B, NCLS), jnp.float32),
        compiler_params=pltpu.CompilerParams(use_tc_tiling_on_sc=False),
        scratch_types=[
            pltpu.VMEM((DC, L), jnp.int32),
            pltpu.VMEM((NBUF, L), jnp.int32),
            pltpu.VMEM((NBUF, L, NCLS), jnp.float32),
            pltpu.VMEM((D_TILE, NCLS), jnp.float32),
            pltpu.SemaphoreType.DMA((NBUF,)),
        ],
    )
    def k(data_hbm, tab_hbm, out_hbm, idx_v, idxq, bufs, out_v, gsem):
        wid = lax.axis_index("s") * NC + lax.axis_index("c")
        doc_base = wid * D_TILE

        offs = tuple(kk * 16 for kk in range(12)) + (L - 16,)

        def xform(d, par):
            for o in offs:
                v = idx_v[d, pl.ds(o, 16)]
                q = lax.shift_left(jnp.bitwise_and(v, N - 1), 3)
                q = jnp.bitwise_or(q, lax.shift_right_logical(v, SH))
                idxq[par, pl.ds(o, 16)] = q

        def fire(d, par):
            xform(d, par)
            pltpu.async_copy(
                tab_hbm.at[idxq.at[par, pl.ds(0, H0)]],
                bufs.at[par].at[pl.ds(0, H0)],
                gsem.at[par],
            )
            pltpu.async_copy(
                tab_hbm.at[idxq.at[par, pl.ds(H0, H1)]],
                bufs.at[par].at[pl.ds(H0, H1)],
                gsem.at[par],
            )

        def wait_par(par):
            pltpu.make_async_copy(
                tab_hbm.at[idxq.at[par, pl.ds(0, H0)]],
                bufs.at[par].at[pl.ds(0, H0)],
                gsem.at[par],
            ).wait()
            pltpu.make_async_copy(
                tab_hbm.at[idxq.at[par, pl.ds(0, H1)]],
                bufs.at[par].at[pl.ds(H0, H1)],
                gsem.at[par],
            ).wait()

        def sum_rows(buf):
            def body(i, a):
                for u in range(8):
                    a = a + buf[i * 8 + u, pl.ds(0, 16)]
                return a

            z = jnp.zeros((16,), jnp.float32)
            return lax.fori_loop(0, L // 8, body, z, unroll=False)

        def chunk_body(c, _):
            pltpu.sync_copy(data_hbm.at[pl.ds(doc_base + c * DC, DC)], idx_v)
            for par in range(NBUF):
                fire(par, par)

            def group_body(p, _):
                for par in range(NBUF):
                    d = NBUF * p + par
                    wait_par(par)
                    a = sum_rows(bufs.at[par])
                    nd = d + NBUF

                    @pl.when(nd < DC)
                    def _():
                        fire(nd, par)

                    out_v[c * DC + d, pl.ds(0, 16)] = a
                return 0

            lax.fori_loop(0, DC // NBUF, group_body, 0, unroll=False)
            return 0

        lax.fori_loop(0, NCHUNK, chunk_body, 0, unroll=False)
        pltpu.sync_copy(out_v, out_hbm.at[pl.ds(wid * D_TILE, D_TILE)])

    return k(data, table)


def _tc_tail(pooled, len_f, b2):
    def body(p_ref, l_ref, b_ref, o_ref):
        o_ref[...] = p_ref[...] / l_ref[...] + b_ref[...]

    return pl.pallas_call(
        body,
        out_shape=jax.ShapeDtypeStruct((B, NCLS), jnp.float32),
    )(pooled, len_f, b2)


def kernel(data, length, embed, W, b):
    len_f = length.astype(jnp.float32).reshape(B, 1)
    packed = _tc_proj_pack(embed.T, W)
    table = packed.reshape(8 * N, NCLS)
    pooled = _sc_pool(data.astype(jnp.int32), table)
    return _tc_tail(pooled, len_f, b.reshape(1, NCLS))


# speedup vs baseline: 3.8981x; 1.1263x over previous
import functools

import jax
import jax.numpy as jnp
from jax import lax
from jax.experimental import pallas as pl
from jax.experimental.pallas import tpu as pltpu
from jax.experimental.pallas import tpu_sc as plsc

B = 16384
L = 200
EMB = 32
NCLS = 16
V = 1000002

NC = 2
NS = 16
NW = NC * NS
D_TILE = B // NW
H0 = 104
H1 = L - H0
NCHUNK = 2
DC = D_TILE // NCHUNK
NBUF = 16

N = 131072
SH = 17
RB = 2048
NBI = N // RB
CMAX = (V - 1) // RB


def _tc_proj_pack(embed_t, W):
    def body(*refs):
        x_refs, w_ref, o_ref = refs[:8], refs[8], refs[9]
        xcat = jnp.concatenate([x_refs[j][...] for j in range(8)], axis=0)
        o_ref[...] = lax.dot_general(
            xcat,
            w_ref[...],
            dimension_numbers=(((0,), (0,)), ((), ())),
            preferred_element_type=jnp.float32,
        )

    def _xspec(j):
        return pl.BlockSpec(
            (EMB, RB), lambda i, j=j: (0, jnp.minimum(j * NBI + i, CMAX))
        )

    wbig = jnp.kron(jnp.eye(8, dtype=jnp.float32), W.T)
    return pl.pallas_call(
        body,
        grid=(NBI,),
        in_specs=[_xspec(j) for j in range(8)]
        + [pl.BlockSpec((8 * EMB, 128), lambda i: (0, 0))],
        out_specs=pl.BlockSpec((RB, 128), lambda i: (i, 0)),
        out_shape=jax.ShapeDtypeStruct((N, 128), jnp.float32),
    )(*([embed_t] * 8), wbig)


def _sc_pool(data, table):
    mesh = plsc.VectorSubcoreMesh(core_axis_name="c", subcore_axis_name="s")

    @functools.partial(
        pl.kernel,
        mesh=mesh,
        out_type=jax.ShapeDtypeStruct((B, NCLS), jnp.float32),
        compiler_params=pltpu.CompilerParams(use_tc_tiling_on_sc=False),
        scratch_types=[
            pltpu.VMEM((DC, L), jnp.int32),
            pltpu.VMEM((NBUF, L), jnp.int32),
            pltpu.VMEM((NBUF, L, NCLS), jnp.float32),
            pltpu.VMEM((D_TILE, NCLS), jnp.float32),
            pltpu.SemaphoreType.DMA((NBUF,)),
        ],
    )
    def k(data_hbm, tab_hbm, out_hbm, idx_v, idxq, bufs, out_v, gsem):
        wid = lax.axis_index("s") * NC + lax.axis_index("c")
        doc_base = wid * D_TILE

        offs = tuple(kk * 16 for kk in range(12)) + (L - 16,)

        def xform(d, par):
            for o in offs:
                v = idx_v[d, pl.ds(o, 16)]
                q = lax.shift_left(jnp.bitwise_and(v, N - 1), 3)
                q = jnp.bitwise_or(q, lax.shift_right_logical(v, SH))
                idxq[par, pl.ds(o, 16)] = q

        def fire(d, par):
            xform(d, par)
            pltpu.async_copy(
                tab_hbm.at[idxq.at[par, pl.ds(0, H0)]],
                bufs.at[par].at[pl.ds(0, H0)],
                gsem.at[par],
            )
            pltpu.async_copy(
                tab_hbm.at[idxq.at[par, pl.ds(H0, H1)]],
                bufs.at[par].at[pl.ds(H0, H1)],
                gsem.at[par],
            )

        def wait_par(par):
            pltpu.make_async_copy(
                tab_hbm.at[idxq.at[par, pl.ds(0, H0)]],
                bufs.at[par].at[pl.ds(0, H0)],
                gsem.at[par],
            ).wait()
            pltpu.make_async_copy(
                tab_hbm.at[idxq.at[par, pl.ds(0, H1)]],
                bufs.at[par].at[pl.ds(H0, H1)],
                gsem.at[par],
            ).wait()

        def sum_rows(buf):
            def body(i, accs):
                a0, a1, a2, a3 = accs
                r = i * 8
                a0 = a0 + buf[r + 0, pl.ds(0, 16)]
                a1 = a1 + buf[r + 1, pl.ds(0, 16)]
                a2 = a2 + buf[r + 2, pl.ds(0, 16)]
                a3 = a3 + buf[r + 3, pl.ds(0, 16)]
                a0 = a0 + buf[r + 4, pl.ds(0, 16)]
                a1 = a1 + buf[r + 5, pl.ds(0, 16)]
                a2 = a2 + buf[r + 6, pl.ds(0, 16)]
                a3 = a3 + buf[r + 7, pl.ds(0, 16)]
                return a0, a1, a2, a3

            z = jnp.zeros((16,), jnp.float32)
            a0, a1, a2, a3 = lax.fori_loop(
                0, L // 8, body, (z, z, z, z), unroll=False
            )
            return (a0 + a1) + (a2 + a3)

        def chunk_body(c, _):
            pltpu.sync_copy(data_hbm.at[pl.ds(doc_base + c * DC, DC)], idx_v)
            for par in range(NBUF):
                fire(par, par)

            def group_body(p, _):
                for par in range(NBUF):
                    d = NBUF * p + par
                    wait_par(par)
                    a = sum_rows(bufs.at[par])
                    nd = d + NBUF

                    @pl.when(nd < DC)
                    def _():
                        fire(nd, par)

                    out_v[c * DC + d, pl.ds(0, 16)] = a
                return 0

            lax.fori_loop(0, DC // NBUF, group_body, 0, unroll=False)
            return 0

        lax.fori_loop(0, NCHUNK, chunk_body, 0, unroll=False)
        pltpu.sync_copy(out_v, out_hbm.at[pl.ds(wid * D_TILE, D_TILE)])

    return k(data, table)


def _tc_tail(pooled, len_f, b2):
    def body(p_ref, l_ref, b_ref, o_ref):
        o_ref[...] = p_ref[...] / l_ref[...] + b_ref[...]

    return pl.pallas_call(
        body,
        out_shape=jax.ShapeDtypeStruct((B, NCLS), jnp.float32),
    )(pooled, len_f, b2)


def kernel(data, length, embed, W, b):
    len_f = length.astype(jnp.float32).reshape(B, 1)
    packed = _tc_proj_pack(embed.T, W)
    table = packed.reshape(8 * N, NCLS)
    pooled = _sc_pool(data.astype(jnp.int32), table)
    return _tc_tail(pooled, len_f, b.reshape(1, NCLS))


# speedup vs baseline: 4.0019x; 1.0266x over previous
import functools

import jax
import jax.numpy as jnp
from jax import lax
from jax.experimental import pallas as pl
from jax.experimental.pallas import tpu as pltpu
from jax.experimental.pallas import tpu_sc as plsc

B = 16384
L = 200
EMB = 32
NCLS = 16
V = 1000002

NC = 2
NS = 16
NW = NC * NS
D_TILE = B // NW
H0 = 104
H1 = L - H0
NCHUNK = 2
DC = D_TILE // NCHUNK
NBUF = 16

N = 131072
SH = 17
RB = 2048
NBI = N // RB
CMAX = (V - 1) // RB


def _tc_proj_pack(embed_t, W):
    def body(*refs):
        x_refs, w_ref, o_ref = refs[:8], refs[8], refs[9]
        xcat = jnp.concatenate([x_refs[j][...] for j in range(8)], axis=0)
        o_ref[...] = lax.dot_general(
            xcat,
            w_ref[...],
            dimension_numbers=(((0,), (0,)), ((), ())),
            preferred_element_type=jnp.float32,
        )

    def _xspec(j):
        return pl.BlockSpec(
            (EMB, RB), lambda i, j=j: (0, jnp.minimum(j * NBI + i, CMAX))
        )

    wbig = jnp.kron(jnp.eye(8, dtype=jnp.float32), W.T)
    return pl.pallas_call(
        body,
        grid=(NBI,),
        in_specs=[_xspec(j) for j in range(8)]
        + [pl.BlockSpec((8 * EMB, 128), lambda i: (0, 0))],
        out_specs=pl.BlockSpec((RB, 128), lambda i: (i, 0)),
        out_shape=jax.ShapeDtypeStruct((N, 128), jnp.float32),
    )(*([embed_t] * 8), wbig)


def _sc_pool(data, table, length, bias):
    mesh = plsc.VectorSubcoreMesh(core_axis_name="c", subcore_axis_name="s")

    @functools.partial(
        pl.kernel,
        mesh=mesh,
        out_type=jax.ShapeDtypeStruct((B, NCLS), jnp.float32),
        compiler_params=pltpu.CompilerParams(use_tc_tiling_on_sc=False),
        scratch_types=[
            pltpu.VMEM((DC, L), jnp.int32),
            pltpu.VMEM((NBUF, L), jnp.int32),
            pltpu.VMEM((NBUF, L, NCLS), jnp.float32),
            pltpu.VMEM((D_TILE, NCLS), jnp.float32),
            pltpu.VMEM((D_TILE,), jnp.int32),
            pltpu.VMEM((NCLS,), jnp.float32),
            pltpu.SemaphoreType.DMA((NBUF,)),
        ],
    )
    def k(data_hbm, tab_hbm, len_hbm, bias_hbm, out_hbm, idx_v, idxq, bufs,
          out_t, len_v, bias_v, gsem):
        wid = lax.axis_index("s") * NC + lax.axis_index("c")
        doc_base = wid * D_TILE
        cls_iota = lax.iota(jnp.int32, 16)

        pltpu.sync_copy(len_hbm.at[pl.ds(doc_base, D_TILE)], len_v)
        pltpu.sync_copy(bias_hbm, bias_v)

        offs = tuple(kk * 16 for kk in range(12)) + (L - 16,)

        def xform(d, par):
            for o in offs:
                v = idx_v[d, pl.ds(o, 16)]
                q = lax.shift_left(jnp.bitwise_and(v, N - 1), 3)
                q = jnp.bitwise_or(q, lax.shift_right_logical(v, SH))
                idxq[par, pl.ds(o, 16)] = q

        def fire(d, par):
            xform(d, par)
            pltpu.async_copy(
                tab_hbm.at[idxq.at[par, pl.ds(0, H0)]],
                bufs.at[par].at[pl.ds(0, H0)],
                gsem.at[par],
            )
            pltpu.async_copy(
                tab_hbm.at[idxq.at[par, pl.ds(H0, H1)]],
                bufs.at[par].at[pl.ds(H0, H1)],
                gsem.at[par],
            )

        def wait_par(par):
            pltpu.make_async_copy(
                tab_hbm.at[idxq.at[par, pl.ds(0, H0)]],
                bufs.at[par].at[pl.ds(0, H0)],
                gsem.at[par],
            ).wait()
            pltpu.make_async_copy(
                tab_hbm.at[idxq.at[par, pl.ds(0, H1)]],
                bufs.at[par].at[pl.ds(H0, H1)],
                gsem.at[par],
            ).wait()

        def sum_rows(buf):
            def body(i, accs):
                a0, a1, a2, a3 = accs
                r = i * 8
                a0 = a0 + buf[r + 0, pl.ds(0, 16)]
                a1 = a1 + buf[r + 1, pl.ds(0, 16)]
                a2 = a2 + buf[r + 2, pl.ds(0, 16)]
                a3 = a3 + buf[r + 3, pl.ds(0, 16)]
                a0 = a0 + buf[r + 4, pl.ds(0, 16)]
                a1 = a1 + buf[r + 5, pl.ds(0, 16)]
                a2 = a2 + buf[r + 6, pl.ds(0, 16)]
                a3 = a3 + buf[r + 7, pl.ds(0, 16)]
                return a0, a1, a2, a3

            z = jnp.zeros((16,), jnp.float32)
            a0, a1, a2, a3 = lax.fori_loop(
                0, L // 8, body, (z, z, z, z), unroll=False
            )
            return (a0 + a1) + (a2 + a3)

        def chunk_body(c, _):
            pltpu.sync_copy(data_hbm.at[pl.ds(doc_base + c * DC, DC)], idx_v)
            for par in range(NBUF):
                fire(par, par)

            def group_body(p, _):
                lvec = len_v[pl.ds(c * DC + p * NBUF, 16)].astype(jnp.float32)
                for par in range(NBUF):
                    d = NBUF * p + par
                    wait_par(par)
                    a = sum_rows(bufs.at[par])
                    nd = d + NBUF

                    @pl.when(nd < DC)
                    def _():
                        fire(nd, par)

                    row = c * DC + d
                    lb = lvec[jnp.full((16,), par, jnp.int32)]
                    val = a / lb + bias_v[pl.ds(0, 16)]
                    out_t[row, pl.ds(0, 16)] = val
                return 0

            lax.fori_loop(0, DC // NBUF, group_body, 0, unroll=False)
            return 0

        lax.fori_loop(0, NCHUNK, chunk_body, 0, unroll=False)
        pltpu.sync_copy(out_t, out_hbm.at[pl.ds(wid * D_TILE, D_TILE)])

    return k(data, table, length, bias)


def kernel(data, length, embed, W, b):
    packed = _tc_proj_pack(embed.T, W)
    table = packed.reshape(8 * N, NCLS)
    return _sc_pool(data.astype(jnp.int32), table, length.astype(jnp.int32), b)


# speedup vs baseline: 4.3580x; 1.0890x over previous
import functools

import jax
import jax.numpy as jnp
from jax import lax
from jax.experimental import pallas as pl
from jax.experimental.pallas import tpu as pltpu
from jax.experimental.pallas import tpu_sc as plsc

B = 16384
L = 200
EMB = 32
NCLS = 16
V = 1000002

NC = 2
NS = 16
NW = NC * NS
D_TILE = B // NW
H0 = 104
H1 = L - H0
NCHUNK = 2
DC = D_TILE // NCHUNK
NBUF = 16

N = 131072
SH = 17
RB = 8192
NBI = N // RB
CMAX = (V - 1) // RB


def _tc_proj_pack(embed_t, W):
    def body(*refs):
        x_refs, w_ref, o_ref = refs[:8], refs[8], refs[9]
        xcat = jnp.concatenate([x_refs[j][...] for j in range(8)], axis=0)
        o_ref[...] = lax.dot_general(
            xcat,
            w_ref[...],
            dimension_numbers=(((0,), (0,)), ((), ())),
            preferred_element_type=jnp.float32,
        )

    def _xspec(j):
        return pl.BlockSpec(
            (EMB, RB), lambda i, j=j: (0, jnp.minimum(j * NBI + i, CMAX))
        )

    wbig = jnp.kron(jnp.eye(8, dtype=jnp.float32), W.T)
    return pl.pallas_call(
        body,
        grid=(NBI,),
        in_specs=[_xspec(j) for j in range(8)]
        + [pl.BlockSpec((8 * EMB, 128), lambda i: (0, 0))],
        out_specs=pl.BlockSpec((RB, 128), lambda i: (i, 0)),
        out_shape=jax.ShapeDtypeStruct((N, 128), jnp.float32),
    )(*([embed_t] * 8), wbig)


def _sc_pool(data, table, length, bias):
    mesh = plsc.VectorSubcoreMesh(core_axis_name="c", subcore_axis_name="s")

    @functools.partial(
        pl.kernel,
        mesh=mesh,
        out_type=jax.ShapeDtypeStruct((B, NCLS), jnp.float32),
        compiler_params=pltpu.CompilerParams(use_tc_tiling_on_sc=False),
        scratch_types=[
            pltpu.VMEM((DC, L), jnp.int32),
            pltpu.VMEM((NBUF, L), jnp.int32),
            pltpu.VMEM((NBUF, L, NCLS), jnp.float32),
            pltpu.VMEM((D_TILE, NCLS), jnp.float32),
            pltpu.VMEM((D_TILE,), jnp.int32),
            pltpu.VMEM((NCLS,), jnp.float32),
            pltpu.SemaphoreType.DMA((NBUF,)),
        ],
    )
    def k(data_hbm, tab_hbm, len_hbm, bias_hbm, out_hbm, idx_v, idxq, bufs,
          out_t, len_v, bias_v, gsem):
        wid = lax.axis_index("s") * NC + lax.axis_index("c")
        doc_base = wid * D_TILE
        cls_iota = lax.iota(jnp.int32, 16)

        pltpu.sync_copy(len_hbm.at[pl.ds(doc_base, D_TILE)], len_v)
        pltpu.sync_copy(bias_hbm, bias_v)

        offs = tuple(kk * 16 for kk in range(12)) + (L - 16,)

        def xform(d, par):
            for o in offs:
                v = idx_v[d, pl.ds(o, 16)]
                q = lax.shift_left(jnp.bitwise_and(v, N - 1), 3)
                q = jnp.bitwise_or(q, lax.shift_right_logical(v, SH))
                idxq[par, pl.ds(o, 16)] = q

        def fire(d, par):
            xform(d, par)
            pltpu.async_copy(
                tab_hbm.at[idxq.at[par, pl.ds(0, H0)]],
                bufs.at[par].at[pl.ds(0, H0)],
                gsem.at[par],
            )
            pltpu.async_copy(
                tab_hbm.at[idxq.at[par, pl.ds(H0, H1)]],
                bufs.at[par].at[pl.ds(H0, H1)],
                gsem.at[par],
            )

        def wait_par(par):
            pltpu.make_async_copy(
                tab_hbm.at[idxq.at[par, pl.ds(0, H0)]],
                bufs.at[par].at[pl.ds(0, H0)],
                gsem.at[par],
            ).wait()
            pltpu.make_async_copy(
                tab_hbm.at[idxq.at[par, pl.ds(0, H1)]],
                bufs.at[par].at[pl.ds(H0, H1)],
                gsem.at[par],
            ).wait()

        def sum_rows(buf):
            def body(i, accs):
                a0, a1, a2, a3 = accs
                r = i * 8
                a0 = a0 + buf[r + 0, pl.ds(0, 16)]
                a1 = a1 + buf[r + 1, pl.ds(0, 16)]
                a2 = a2 + buf[r + 2, pl.ds(0, 16)]
                a3 = a3 + buf[r + 3, pl.ds(0, 16)]
                a0 = a0 + buf[r + 4, pl.ds(0, 16)]
                a1 = a1 + buf[r + 5, pl.ds(0, 16)]
                a2 = a2 + buf[r + 6, pl.ds(0, 16)]
                a3 = a3 + buf[r + 7, pl.ds(0, 16)]
                return a0, a1, a2, a3

            z = jnp.zeros((16,), jnp.float32)
            a0, a1, a2, a3 = lax.fori_loop(
                0, L // 8, body, (z, z, z, z), unroll=False
            )
            return (a0 + a1) + (a2 + a3)

        def chunk_body(c, _):
            pltpu.sync_copy(data_hbm.at[pl.ds(doc_base + c * DC, DC)], idx_v)
            for par in range(NBUF):
                fire(par, par)

            def group_body(p, _):
                lvec = len_v[pl.ds(c * DC + p * NBUF, 16)].astype(jnp.float32)
                for par in range(NBUF):
                    d = NBUF * p + par
                    wait_par(par)
                    a = sum_rows(bufs.at[par])
                    nd = d + NBUF

                    @pl.when(nd < DC)
                    def _():
                        fire(nd, par)

                    row = c * DC + d
                    lb = lvec[jnp.full((16,), par, jnp.int32)]
                    val = a / lb + bias_v[pl.ds(0, 16)]
                    out_t[row, pl.ds(0, 16)] = val
                return 0

            lax.fori_loop(0, DC // NBUF, group_body, 0, unroll=False)
            return 0

        lax.fori_loop(0, NCHUNK, chunk_body, 0, unroll=False)
        pltpu.sync_copy(out_t, out_hbm.at[pl.ds(wid * D_TILE, D_TILE)])

    return k(data, table, length, bias)


def kernel(data, length, embed, W, b):
    packed = _tc_proj_pack(embed.T, W)
    table = packed.reshape(8 * N, NCLS)
    return _sc_pool(data.astype(jnp.int32), table, length.astype(jnp.int32), b)
